# baseline (speedup 1.0000x reference)
import dataclasses
import functools

import jax
import jax.numpy as jnp
from jax.experimental import pallas as pl
from jax.experimental.pallas import tpu as pltpu
from jax.experimental.pallas import tpu_sc as plsc

_K = 100
_N_TOKENS = 8192
_DIM = 2048
_BLK = 256
_NEG_INF = float("-inf")


def _fused_body(x_ref, we_ref, be_ref, wd_ref, bd_ref,
                sparse_ref, recon_ref, stats_ref):
    i = pl.program_id(0)

    @pl.when(i == 0)
    def _init():
        init = jnp.where(
            jax.lax.broadcasted_iota(jnp.int32, (8, 128), 0) == 1,
            _NEG_INF, 0.0).astype(jnp.float32)
        stats_ref[...] = init

    x_blk = x_ref[...]
    xb = x_blk.astype(jnp.bfloat16)
    we = we_ref[...]
    act_t = jax.lax.dot_general(
        we, xb, (((1,), (1,)), ((), ())),
        preferred_element_type=jnp.float32)
    act_t = act_t + be_ref[...]

    abits = jax.lax.bitcast_convert_type(act_t, jnp.int32) & jnp.int32(
        0x7FFFFFFF)

    kf = jnp.float32(_K)

    def _count_ge(th):
        return jnp.sum(
            jnp.where(abits >= th, 1.0, 0.0).astype(jnp.float32),
            axis=0, keepdims=True)

    def _val_step(b, t):
        cand = t | jnp.left_shift(jnp.int32(1), 30 - b)
        cnt = _count_ge(cand)
        return jnp.where(cnt >= kf, cand, t)

    t0 = jnp.zeros((1, _BLK), jnp.int32)
    t = jax.lax.fori_loop(0, 31, _val_step, t0)

    n_gt = _count_ge(t + 1)
    need = kf - n_gt
    eq = abits == t
    col = jax.lax.broadcasted_iota(jnp.int32, (_DIM, _BLK), 0)

    def _idx_step(b, J):
        cand = J | jnp.left_shift(jnp.int32(1), 10 - b)
        c = jnp.sum(
            jnp.where(eq & (col < cand), 1.0, 0.0).astype(jnp.float32),
            axis=0, keepdims=True)
        return jnp.where(c < need, cand, J)

    J = jax.lax.fori_loop(0, 11, _idx_step, jnp.zeros((1, _BLK), jnp.int32))
    mask = (abits > t) | (eq & (col <= J))

    sparse_t = jnp.where(mask, act_t, 0.0)
    sparse_ref[...] = sparse_t.T

    recon_t = jax.lax.dot_general(
        wd_ref[...], sparse_t.astype(jnp.bfloat16),
        (((1,), (0,)), ((), ())),
        preferred_element_type=jnp.float32) + bd_ref[...]
    recon_ref[...] = recon_t.T

    absel = jnp.where(mask, jnp.abs(act_t), 0.0)
    l0 = jnp.sum(jnp.where(mask & (abits != 0), 1.0, 0.0).astype(jnp.float32),
                 axis=0, keepdims=True)
    dev = l0 - kf
    s_dev = jnp.sum(dev)
    s_dev2 = jnp.sum(dev * dev)
    s_l0 = jnp.sum(l0)
    s_abs = jnp.sum(absel)
    m_abs = jnp.max(jnp.where(mask & (abits != 0), jnp.abs(act_t), _NEG_INF))

    row = jax.lax.broadcasted_iota(jnp.int32, (8, 128), 0)
    lane = jax.lax.broadcasted_iota(jnp.int32, (8, 128), 1)
    sum_part = jnp.where(
        (row == 0) & (lane == 0), s_dev,
        jnp.where((row == 0) & (lane == 1), s_dev2,
                  jnp.where((row == 0) & (lane == 2), s_abs,
                            jnp.where((row == 0) & (lane == 3), s_l0, 0.0))))
    max_part = jnp.where((row == 1) & (lane == 0), m_abs, _NEG_INF)
    stats_ref[...] = jnp.maximum(stats_ref[...] + sum_part, max_part)


@functools.partial(jax.jit, static_argnames=())
def _fused(x, we_bf, be2, wd_bf, bd2):
    grid = _N_TOKENS // _BLK
    return pl.pallas_call(
        _fused_body,
        grid=(grid,),
        in_specs=[
            pl.BlockSpec((_BLK, _DIM), lambda i: (i, 0)),
            pl.BlockSpec((_DIM, _DIM), lambda i: (0, 0)),
            pl.BlockSpec((_DIM, 1), lambda i: (0, 0)),
            pl.BlockSpec((_DIM, _DIM), lambda i: (0, 0)),
            pl.BlockSpec((_DIM, 1), lambda i: (0, 0)),
        ],
        out_specs=[
            pl.BlockSpec((_BLK, _DIM), lambda i: (i, 0)),
            pl.BlockSpec((_BLK, _DIM), lambda i: (i, 0)),
            pl.BlockSpec((8, 128), lambda i: (0, 0)),
        ],
        out_shape=[
            jax.ShapeDtypeStruct((_N_TOKENS, _DIM), jnp.float32),
            jax.ShapeDtypeStruct((_N_TOKENS, _DIM), jnp.float32),
            jax.ShapeDtypeStruct((8, 128), jnp.float32),
        ],
    )(x, we_bf, be2, wd_bf, bd2)


_NB = 65536
_NW = 32
_ELEMS = _N_TOKENS * _DIM
_PER_W = _ELEMS // _NW
_CHUNK = 32768
_NCHUNK = _PER_W // _CHUNK


def _sc_compiler_params():
    cp = pltpu.CompilerParams()
    if "needs_layout_passes" in pltpu.CompilerParams.__dataclass_fields__:
        cp = dataclasses.replace(cp, needs_layout_passes=False)
    return cp


def _sc_hist(sparse_flat):
    mesh = plsc.VectorSubcoreMesh(core_axis_name="c", subcore_axis_name="s")

    @functools.partial(
        pl.kernel, mesh=mesh,
        out_type=jax.ShapeDtypeStruct((_NW, _NB), jnp.float32),
        scratch_types=[
            pltpu.VMEM((_CHUNK,), jnp.float32),
            pltpu.VMEM((_NB,), jnp.float32),
        ],
        compiler_params=_sc_compiler_params(),
    )
    def hist_kernel(flat_hbm, hist_hbm, chunk_ref, hist_ref):
        wid = jax.lax.axis_index("s") * 2 + jax.lax.axis_index("c")

        @pl.loop(0, _NB, step=16)
        def _zero(i):
            hist_ref[pl.ds(i, 16)] = jnp.zeros((16,), jnp.float32)

        base = wid * _PER_W
        ones = jnp.ones((16,), jnp.float32)

        @pl.loop(0, _NCHUNK)
        def _chunk(g):
            pltpu.sync_copy(flat_hbm.at[pl.ds(base + g * _CHUNK, _CHUNK)],
                            chunk_ref)

            @pl.loop(0, _CHUNK, step=16)
            def _vec(i):
                v = chunk_ref[pl.ds(i, 16)]
                bits = plsc.bitcast(v, jnp.int32) & jnp.int32(0x7FFFFFFF)
                bucket = jax.lax.shift_right_logical(bits, 15)
                plsc.addupdate_scatter(hist_ref, [bucket], ones,
                                       mask=bits != 0)

        pltpu.sync_copy(hist_ref, hist_hbm.at[wid])

    return hist_kernel(sparse_flat)


def _median_body(hist_ref, out_ref):
    h = hist_ref[...]
    tot = jnp.sum(h, axis=0, keepdims=True)
    n = jnp.sum(tot)
    iota = jax.lax.broadcasted_iota(jnp.int32, (1, _NB), 1)
    pos = 0.5 * (n - 1.0)
    r0 = jnp.floor(pos)
    frac = pos - r0
    r1 = jnp.ceil(pos)

    def _find(r):
        def step(b, J):
            cand = J | jnp.left_shift(jnp.int32(1), 15 - b)
            cnt = jnp.sum(jnp.where(iota < cand, tot, 0.0))
            return jnp.where(cnt <= r, cand, J)

        return jax.lax.fori_loop(0, 16, step, jnp.int32(0))

    j0 = _find(r0)
    j1 = _find(r1)
    row = jax.lax.broadcasted_iota(jnp.int32, (8, 128), 0)
    lane = jax.lax.broadcasted_iota(jnp.int32, (8, 128), 1)
    at00 = (row == 0) & (lane == 0)
    v0 = jax.lax.bitcast_convert_type(
        jnp.where(at00, jnp.left_shift(j0, 15) | jnp.int32(0x4000), 0),
        jnp.float32)
    v1 = jax.lax.bitcast_convert_type(
        jnp.where(at00, jnp.left_shift(j1, 15) | jnp.int32(0x4000), 0),
        jnp.float32)
    out_ref[...] = v0 * (1.0 - frac) + v1 * frac


def _median_from_hists(hists):
    out = pl.pallas_call(
        _median_body,
        in_specs=[pl.BlockSpec((_NW, _NB), lambda: (0, 0))],
        out_specs=pl.BlockSpec((8, 128), lambda: (0, 0)),
        out_shape=jax.ShapeDtypeStruct((8, 128), jnp.float32),
    )(hists)
    return out[0, 0]


def kernel(x, W_enc, b_enc, W_dec, b_dec):
    we_bf = W_enc.astype(jnp.bfloat16)
    wd_bf = W_dec.astype(jnp.bfloat16)
    be2 = b_enc.reshape(_DIM, 1)
    bd2 = b_dec.reshape(_DIM, 1)

    recon_sparse = _fused(x, we_bf, be2, wd_bf, bd2)
    sparse, recon, stats = recon_sparse

    n = jnp.float32(_N_TOKENS)
    s_dev = stats[0, 0]
    s_dev2 = stats[0, 1]
    s_abs = stats[0, 2]
    n_active = stats[0, 3]
    max_activation = stats[1, 0]
    l0_mean = jnp.float32(_K) + s_dev / n
    var = (s_dev2 - s_dev * s_dev / n) / (n - 1.0)
    l0_std = jnp.sqrt(jnp.maximum(var, 0.0))
    mean_activation = s_abs / n_active

    hists = _sc_hist(sparse.reshape(-1))
    median_activation = _median_from_hists(hists)

    return (recon, sparse, l0_mean, l0_std, mean_activation,
            max_activation, median_activation)


# speedup vs baseline: 1.2759x; 1.2759x over previous
import dataclasses
import functools

import jax
import jax.numpy as jnp
from jax.experimental import pallas as pl
from jax.experimental.pallas import tpu as pltpu
from jax.experimental.pallas import tpu_sc as plsc

_K = 100
_N_TOKENS = 8192
_DIM = 2048
_BLK = 256
_NEG_INF = float("-inf")


def _fused_body(x_ref, we_ref, be_ref,
                sparse_ref, spb_ref, stats_ref):
    i = pl.program_id(0)

    @pl.when(i == 0)
    def _init():
        init = jnp.where(
            jax.lax.broadcasted_iota(jnp.int32, (8, 128), 0) == 1,
            _NEG_INF, 0.0).astype(jnp.float32)
        stats_ref[...] = init

    x_blk = x_ref[...]
    xb = x_blk.astype(jnp.bfloat16)
    we = we_ref[...]
    act_t = jax.lax.dot_general(
        we, xb, (((1,), (1,)), ((), ())),
        preferred_element_type=jnp.float32)
    act_t = act_t + be_ref[...]

    abits = jax.lax.bitcast_convert_type(act_t, jnp.int32) & jnp.int32(
        0x7FFFFFFF)

    kf = jnp.float32(_K)

    def _count_ge(th):
        return jnp.sum(
            jnp.where(abits >= th, 1.0, 0.0).astype(jnp.float32),
            axis=0, keepdims=True)

    def _val_step(b, t):
        cand = t | jnp.left_shift(jnp.int32(1), 30 - b)
        cnt = _count_ge(cand)
        return jnp.where(cnt >= kf, cand, t)

    t0 = jnp.zeros((1, _BLK), jnp.int32)
    t = jax.lax.fori_loop(0, 31, _val_step, t0)

    n_gt = _count_ge(t + 1)
    need = kf - n_gt
    eq = abits == t
    col = jax.lax.broadcasted_iota(jnp.int32, (_DIM, _BLK), 0)

    def _idx_step(b, J):
        cand = J | jnp.left_shift(jnp.int32(1), 10 - b)
        c = jnp.sum(
            jnp.where(eq & (col < cand), 1.0, 0.0).astype(jnp.float32),
            axis=0, keepdims=True)
        return jnp.where(c < need, cand, J)

    J = jax.lax.fori_loop(0, 11, _idx_step, jnp.zeros((1, _BLK), jnp.int32))
    mask = (abits > t) | (eq & (col <= J))

    sparse_t = jnp.where(mask, act_t, 0.0)
    sparse_ref[...] = sparse_t.T
    spb_ref[...] = sparse_t.astype(jnp.bfloat16).T

    absel = jnp.where(mask, jnp.abs(act_t), 0.0)
    l0 = jnp.sum(jnp.where(absel > 0.0, 1.0, 0.0),
                 axis=0, keepdims=True)
    dev = l0 - kf
    s_dev = jnp.sum(dev)
    s_dev2 = jnp.sum(dev * dev)
    s_l0 = jnp.sum(l0)
    s_abs = jnp.sum(absel)
    m_abs = jnp.max(jnp.where(absel > 0.0, absel, _NEG_INF))

    row = jax.lax.broadcasted_iota(jnp.int32, (8, 128), 0)
    lane = jax.lax.broadcasted_iota(jnp.int32, (8, 128), 1)
    sum_part = jnp.where(
        (row == 0) & (lane == 0), s_dev,
        jnp.where((row == 0) & (lane == 1), s_dev2,
                  jnp.where((row == 0) & (lane == 2), s_abs,
                            jnp.where((row == 0) & (lane == 3), s_l0, 0.0))))
    max_part = jnp.where((row == 1) & (lane == 0), m_abs, _NEG_INF)
    stats_ref[...] = jnp.maximum(stats_ref[...] + sum_part, max_part)


def _encode_select(x, we_bf, be2):
    grid = _N_TOKENS // _BLK
    return pl.pallas_call(
        _fused_body,
        grid=(grid,),
        in_specs=[
            pl.BlockSpec((_BLK, _DIM), lambda i: (i, 0)),
            pl.BlockSpec((_DIM, _DIM), lambda i: (0, 0)),
            pl.BlockSpec((_DIM, 1), lambda i: (0, 0)),
        ],
        out_specs=[
            pl.BlockSpec((_BLK, _DIM), lambda i: (i, 0)),
            pl.BlockSpec((_BLK, _DIM), lambda i: (i, 0)),
            pl.BlockSpec((8, 128), lambda i: (0, 0)),
        ],
        out_shape=[
            jax.ShapeDtypeStruct((_N_TOKENS, _DIM), jnp.float32),
            jax.ShapeDtypeStruct((_N_TOKENS, _DIM), jnp.bfloat16),
            jax.ShapeDtypeStruct((8, 128), jnp.float32),
        ],
    )(x, we_bf, be2)


_DBLK = 512


def _dec_body(spb_ref, wd_ref, bdr_ref, recon_ref):
    rec = jax.lax.dot_general(
        spb_ref[...], wd_ref[...], (((1,), (1,)), ((), ())),
        preferred_element_type=jnp.float32)
    recon_ref[...] = rec + bdr_ref[...]


def _decode(spb, wd_bf, bd_row):
    grid = _N_TOKENS // _DBLK
    return pl.pallas_call(
        _dec_body,
        grid=(grid,),
        in_specs=[
            pl.BlockSpec((_DBLK, _DIM), lambda i: (i, 0)),
            pl.BlockSpec((_DIM, _DIM), lambda i: (0, 0)),
            pl.BlockSpec((1, _DIM), lambda i: (0, 0)),
        ],
        out_specs=pl.BlockSpec((_DBLK, _DIM), lambda i: (i, 0)),
        out_shape=jax.ShapeDtypeStruct((_N_TOKENS, _DIM), jnp.float32),
    )(spb, wd_bf, bd_row)


_NB = 65536
_NW = 32
_ROWS_W = _N_TOKENS // _NW
_CROWS = 16
_NCHUNK = _ROWS_W // _CROWS


def _sc_compiler_params():
    cp = pltpu.CompilerParams()
    if "needs_layout_passes" in pltpu.CompilerParams.__dataclass_fields__:
        cp = dataclasses.replace(cp, needs_layout_passes=False)
    return cp


def _sc_hist(sparse):
    mesh = plsc.VectorSubcoreMesh(core_axis_name="c", subcore_axis_name="s")

    @functools.partial(
        pl.kernel, mesh=mesh,
        out_type=jax.ShapeDtypeStruct((_NW, _NB), jnp.float32),
        scratch_types=[
            pltpu.VMEM((_CROWS, _DIM), jnp.float32),
            pltpu.VMEM((_NB,), jnp.float32),
        ],
        compiler_params=_sc_compiler_params(),
    )
    def hist_kernel(sparse_hbm, hist_hbm, chunk_ref, hist_ref):
        wid = jax.lax.axis_index("s") * 2 + jax.lax.axis_index("c")

        @pl.loop(0, _NB, step=16)
        def _zero(i):
            hist_ref[pl.ds(i, 16)] = jnp.zeros((16,), jnp.float32)

        base = wid * _ROWS_W
        ones = jnp.ones((16,), jnp.float32)

        @pl.loop(0, _NCHUNK)
        def _chunk(g):
            pltpu.sync_copy(
                sparse_hbm.at[pl.ds(base + g * _CROWS, _CROWS)], chunk_ref)

            @pl.loop(0, _CROWS)
            def _row(r):
                @pl.loop(0, _DIM, step=128)
                def _vec(c):
                    for u in range(8):
                        v = chunk_ref[r, pl.ds(c + u * 16, 16)]
                        bits = (plsc.bitcast(v, jnp.int32)
                                & jnp.int32(0x7FFFFFFF))
                        bucket = jax.lax.shift_right_logical(bits, 15)
                        plsc.addupdate_scatter(hist_ref, [bucket], ones,
                                               mask=bits != 0)

        pltpu.sync_copy(hist_ref, hist_hbm.at[wid])

    return hist_kernel(sparse)


def _median_body(hist_ref, out_ref):
    h = hist_ref[...]
    tot = jnp.sum(h, axis=0, keepdims=True)
    n = jnp.sum(tot)
    iota = jax.lax.broadcasted_iota(jnp.int32, (1, _NB), 1)
    pos = 0.5 * (n - 1.0)
    r0 = jnp.floor(pos)
    frac = pos - r0
    r1 = jnp.ceil(pos)

    def _find(r):
        def step(b, J):
            cand = J | jnp.left_shift(jnp.int32(1), 15 - b)
            cnt = jnp.sum(jnp.where(iota < cand, tot, 0.0))
            return jnp.where(cnt <= r, cand, J)

        return jax.lax.fori_loop(0, 16, step, jnp.int32(0))

    j0 = _find(r0)
    j1 = _find(r1)
    row = jax.lax.broadcasted_iota(jnp.int32, (8, 128), 0)
    lane = jax.lax.broadcasted_iota(jnp.int32, (8, 128), 1)
    at00 = (row == 0) & (lane == 0)
    v0 = jax.lax.bitcast_convert_type(
        jnp.where(at00, jnp.left_shift(j0, 15) | jnp.int32(0x4000), 0),
        jnp.float32)
    v1 = jax.lax.bitcast_convert_type(
        jnp.where(at00, jnp.left_shift(j1, 15) | jnp.int32(0x4000), 0),
        jnp.float32)
    out_ref[...] = v0 * (1.0 - frac) + v1 * frac


def _median_from_hists(hists):
    out = pl.pallas_call(
        _median_body,
        in_specs=[pl.BlockSpec((_NW, _NB), lambda: (0, 0))],
        out_specs=pl.BlockSpec((8, 128), lambda: (0, 0)),
        out_shape=jax.ShapeDtypeStruct((8, 128), jnp.float32),
    )(hists)
    return out[0, 0]


def kernel(x, W_enc, b_enc, W_dec, b_dec):
    we_bf = W_enc.astype(jnp.bfloat16)
    wd_bf = W_dec.astype(jnp.bfloat16)
    be2 = b_enc.reshape(_DIM, 1)
    bd_row = b_dec.reshape(1, _DIM)

    sparse, spb, stats = _encode_select(x, we_bf, be2)
    hists = _sc_hist(sparse)
    recon = _decode(spb, wd_bf, bd_row)

    n = jnp.float32(_N_TOKENS)
    s_dev = stats[0, 0]
    s_dev2 = stats[0, 1]
    s_abs = stats[0, 2]
    n_active = stats[0, 3]
    max_activation = stats[1, 0]
    l0_mean = jnp.float32(_K) + s_dev / n
    var = (s_dev2 - s_dev * s_dev / n) / (n - 1.0)
    l0_std = jnp.sqrt(jnp.maximum(var, 0.0))
    mean_activation = s_abs / n_active
    median_activation = _median_from_hists(hists)

    return (recon, sparse, l0_mean, l0_std, mean_activation,
            max_activation, median_activation)
